# baseline (speedup 1.0000x reference)
import functools

import jax
import jax.numpy as jnp
import numpy as np
from jax.experimental import pallas as pl
from jax.experimental.pallas import tpu as pltpu

N_NODES = 10000
N_EDGES = 160000
AVG_LOG = float(np.log(17.0))
L = 2
T = 5
F_IN = 100
F_OUT = 20


def _mm_kernel(a_ref, b_ref, o_ref):
    o_ref[...] = jnp.dot(a_ref[...], b_ref[...],
                         preferred_element_type=jnp.float32)


def _mm(a, b, blk_m=2048):
    m, k = a.shape
    k2, n = b.shape
    assert k == k2
    grid = (m + blk_m - 1) // blk_m
    pad = grid * blk_m - m
    if pad:
        a = jnp.pad(a, ((0, pad), (0, 0)))
    out = pl.pallas_call(
        _mm_kernel,
        grid=(grid,),
        in_specs=[
            pl.BlockSpec((blk_m, k), lambda i: (i, 0)),
            pl.BlockSpec((k, n), lambda i: (0, 0)),
        ],
        out_specs=pl.BlockSpec((blk_m, n), lambda i: (i, 0)),
        out_shape=jax.ShapeDtypeStruct((grid * blk_m, n), jnp.float32),
    )(a, b)
    return out[:m] if pad else out


def kernel(x, edge_index, edge_attr, node_W, node_b, edge_emb_W, edge_emb_b,
           edge_enc_W, edge_enc_b, pre_W, pre_b, post_W, post_b, lin_W, lin_b,
           bn_gamma, bn_beta, emlp_W1, emlp_b1, emlp_W2, emlp_b2,
           mlp_W1, mlp_b1, mlp_W2, mlp_b2, mlp_W3, mlp_b3):
    src = edge_index[0]
    dst = edge_index[1]
    x = _mm(x, node_W) + node_b
    ea = _mm(edge_attr, edge_emb_W) + edge_emb_b
    cnt = jax.ops.segment_sum(jnp.ones((N_EDGES,), jnp.float32), dst,
                              num_segments=N_NODES)
    deg = jnp.maximum(cnt, 1.0)
    amp = (jnp.log(deg + 1.0) / AVG_LOG)[:, None, None]
    att = (AVG_LOG / jnp.log(deg + 1.0))[:, None, None]
    has = (cnt > 0)[:, None, None]
    denom = jnp.maximum(cnt, 1.0)[:, None, None]
    for i in range(L):
        e = _mm(ea, edge_enc_W[i]) + edge_enc_b[i]
        h = jnp.concatenate([x[dst], x[src], e], axis=-1)
        wflat = jnp.transpose(pre_W[i], (1, 0, 2)).reshape(3 * F_IN, T * F_IN)
        m = _mm(h, wflat).reshape(N_EDGES, T, F_IN) + pre_b[i]
        mean = jax.ops.segment_sum(m, dst, num_segments=N_NODES) / denom
        mean_sq = jax.ops.segment_sum(m * m, dst, num_segments=N_NODES) / denom
        std = jnp.sqrt(jax.nn.relu(mean_sq - mean * mean) + 1e-5)
        mn = jnp.where(has, jax.ops.segment_min(m, dst, num_segments=N_NODES), 0.0)
        mx = jnp.where(has, jax.ops.segment_max(m, dst, num_segments=N_NODES), 0.0)
        agg = jnp.concatenate([mean, mn, mx, std], axis=-1)
        agg = jnp.concatenate([agg, agg * amp, agg * att], axis=-1)
        xt = jnp.broadcast_to(x[:, None, :], (N_NODES, T, F_IN))
        o = jnp.einsum('ntf,tfo->nto', jnp.concatenate([xt, agg], axis=-1),
                       post_W[i]) + post_b[i]
        o = _mm(o.reshape(N_NODES, T * F_OUT), lin_W[i]) + lin_b[i]
        mu = o.mean(axis=0)
        var = o.var(axis=0)
        o = (o - mu) / jnp.sqrt(var + 1e-5) * bn_gamma[i] + bn_beta[i]
        x = (x + jax.nn.relu(o)) / 2.0
        msg = jnp.concatenate([x[src], x[dst], ea], axis=-1)
        msg = jax.nn.relu(_mm(msg, emlp_W1[i]) + emlp_b1[i])
        msg = _mm(msg, emlp_W2[i]) + emlp_b2[i]
        ea = ea + msg / 2.0
    xe = jax.nn.relu(jnp.concatenate([x[src], x[dst]], axis=-1))
    out = jnp.concatenate([xe, ea], axis=-1)
    out = jax.nn.relu(_mm(out, mlp_W1) + mlp_b1)
    out = jax.nn.relu(_mm(out, mlp_W2) + mlp_b2)
    return _mm(out, mlp_W3) + mlp_b3
